# Initial kernel scaffold; baseline (speedup 1.0000x reference)
import jax
import jax.numpy as jnp
from jax.experimental import pallas as pl


def kernel(frame_desc, point_desc, pos, fd_buf, pd_buf, pos_buf, kf_desc_idx, kf_pos_idx):
    raise NotImplementedError("write your pallas kernel here")


# speedup vs baseline: 13.2154x; 13.2154x over previous
import functools

import jax
import jax.numpy as jnp
from jax import lax
from jax.experimental import pallas as pl
from jax.experimental.pallas import tpu as pltpu
from jax.experimental.pallas import tpu_sc as plsc

_B = 16
_NUM_FEA = 1024
_D_POINT = 128
_D_FRAME = 256

_info = plsc.get_sparse_core_info()
_NC = _info.num_cores
_NS = _info.num_subcores
_NW = _NC * _NS

_FD = _B * _D_FRAME
_PD = _B * _NUM_FEA * _D_POINT
_POS = _B * _NUM_FEA * 3


def _store_body(fd_in, pd_in, pos_in, fd_out, pd_out, pos_out):
    wid = lax.axis_index("s") * _NC + lax.axis_index("c")
    pdw = _PD // _NW
    posw = _POS // _NW
    fdw = _FD // _NW
    pltpu.sync_copy(pd_in.at[pl.ds(wid * pdw, pdw)], pd_out.at[pl.ds(wid * pdw, pdw)])
    pltpu.sync_copy(pos_in.at[pl.ds(wid * posw, posw)], pos_out.at[pl.ds(wid * posw, posw)])
    pltpu.sync_copy(fd_in.at[pl.ds(wid * fdw, fdw)], fd_out.at[pl.ds(wid * fdw, fdw)])


_sc_store = functools.partial(
    pl.kernel,
    mesh=plsc.VectorSubcoreMesh(core_axis_name="c", subcore_axis_name="s"),
    out_type=[
        jax.ShapeDtypeStruct((_FD,), jnp.float32),
        jax.ShapeDtypeStruct((_PD,), jnp.float32),
        jax.ShapeDtypeStruct((_POS,), jnp.float32),
    ],
)(_store_body)


def kernel(frame_desc, point_desc, pos, fd_buf, pd_buf, pos_buf, kf_desc_idx, kf_pos_idx):
    out_fd, out_pd, out_pos = _sc_store(
        frame_desc.reshape(_FD), point_desc.reshape(_PD), pos.reshape(_POS))
    return (out_fd.reshape(_B, _D_FRAME),
            out_pd.reshape(_B, _NUM_FEA, _D_POINT),
            out_pos.reshape(_B, _NUM_FEA, 3))
